# Initial kernel scaffold; baseline (speedup 1.0000x reference)
import jax
import jax.numpy as jnp
from jax.experimental import pallas as pl


def kernel(word_ids, sequences, sequences_length, char_embed, W, b):
    raise NotImplementedError("write your pallas kernel here")


# speedup vs baseline: 17.7260x; 17.7260x over previous
import functools

import jax
import jax.numpy as jnp
from jax import lax
from jax.experimental import pallas as pl
from jax.experimental.pallas import tpu as pltpu
from jax.experimental.pallas import tpu_sc as plsc

MAX_WORD_LEN = 16
CHAR_VOCAB = 128
CHAR_DIM = 64
EMBED_SIZE = 128

V_BLOCK = 2000
CHUNK = 128


def _table_body(seqs_ref, lens_ref, ce_ref, w_ref, b_ref, out_ref):
    seqs = seqs_ref[...]
    lens = lens_ref[...]
    iota_c = lax.broadcasted_iota(jnp.int32, (1, CHAR_VOCAB), 1)
    counts = jnp.zeros((V_BLOCK, CHAR_VOCAB), jnp.float32)
    for l in range(MAX_WORD_LEN):
        ch = seqs[:, l:l + 1]
        valid = l < lens
        oh = jnp.logical_and(ch == iota_c, valid)
        counts = counts + oh.astype(jnp.float32)
    inv_len = 1.0 / lens.astype(jnp.float32)
    pooled = jnp.dot(counts, ce_ref[...],
                     preferred_element_type=jnp.float32) * inv_len
    out_ref[...] = jnp.dot(pooled, w_ref[...],
                           preferred_element_type=jnp.float32) + b_ref[...]


@functools.lru_cache(maxsize=None)
def _make_table_builder(vocab):
    grid = vocab // V_BLOCK
    return pl.pallas_call(
        _table_body,
        grid=(grid,),
        in_specs=[
            pl.BlockSpec((V_BLOCK, MAX_WORD_LEN), lambda i: (i, 0)),
            pl.BlockSpec((V_BLOCK, 1), lambda i: (i, 0)),
            pl.BlockSpec((CHAR_VOCAB, CHAR_DIM), lambda i: (0, 0)),
            pl.BlockSpec((CHAR_DIM, EMBED_SIZE), lambda i: (0, 0)),
            pl.BlockSpec((1, EMBED_SIZE), lambda i: (0, 0)),
        ],
        out_specs=pl.BlockSpec((V_BLOCK, EMBED_SIZE), lambda i: (i, 0)),
        out_shape=jax.ShapeDtypeStruct((vocab, EMBED_SIZE), jnp.float32),
    )


@functools.lru_cache(maxsize=None)
def _make_gather(n, vocab):
    info = plsc.get_sparse_core_info()
    nw = info.num_cores * info.num_subcores
    assert n % (nw * CHUNK) == 0
    n_chunks = n // (nw * CHUNK)
    mesh = plsc.VectorSubcoreMesh(core_axis_name="c", subcore_axis_name="s")

    @functools.partial(
        pl.kernel,
        mesh=mesh,
        out_type=jax.ShapeDtypeStruct((n, EMBED_SIZE), jnp.float32),
        scratch_types=[
            pltpu.VMEM((n_chunks, CHUNK), jnp.int32),
            pltpu.VMEM((CHUNK, EMBED_SIZE), jnp.float32),
            pltpu.SemaphoreType.DMA,
        ],
    )
    def gather_kernel(idx_hbm, table_hbm, out_hbm, idx_v, rows_v, sem):
        wid = lax.axis_index("s") * info.num_cores + lax.axis_index("c")
        row_base = wid * n_chunks
        pltpu.sync_copy(idx_hbm.at[wid], idx_v)

        def chunk_body(c, carry):
            pltpu.async_copy(table_hbm.at[idx_v.at[c]], rows_v, sem).wait()
            pltpu.sync_copy(
                rows_v, out_hbm.at[pl.ds((row_base + c) * CHUNK, CHUNK)])
            return carry

        lax.fori_loop(0, n_chunks, chunk_body, 0)

    def run(ids_flat, table):
        return gather_kernel(ids_flat.reshape(nw, n_chunks, CHUNK), table)

    return run


def kernel(word_ids, sequences, sequences_length, char_embed, W, b):
    shape = word_ids.shape
    n = word_ids.size
    vocab = sequences.shape[0]
    ids = jnp.asarray(word_ids, jnp.int32).reshape(-1)
    seqs = jnp.asarray(sequences, jnp.int32)
    lens = jnp.asarray(sequences_length, jnp.int32).reshape(vocab, 1)
    table = _make_table_builder(vocab)(
        seqs, lens, char_embed, W, b.reshape(1, EMBED_SIZE))
    out = _make_gather(n, vocab)(ids, table)
    return out.reshape(shape + (EMBED_SIZE,))


# speedup vs baseline: 29.2817x; 1.6519x over previous
import functools

import jax
import jax.numpy as jnp
from jax import lax
from jax.experimental import pallas as pl
from jax.experimental.pallas import tpu as pltpu
from jax.experimental.pallas import tpu_sc as plsc

MAX_WORD_LEN = 16
CHAR_VOCAB = 128
CHAR_DIM = 64
EMBED_SIZE = 128

V_BLOCK = 2000
CHUNK = 128


def _table_body(seqs_ref, lens_ref, ce_ref, w_ref, b_ref, out_ref):
    L = MAX_WORD_LEN
    seqs = seqs_ref[...]
    lens = lens_ref[...]
    lens_f = lens.astype(jnp.float32)
    ones_l = jnp.ones((1, L), jnp.float32)
    lens_b = jnp.dot(lens_f, ones_l,
                     preferred_element_type=jnp.float32)
    pos = lax.broadcasted_iota(jnp.int32, (1, L), 1).astype(jnp.float32)
    sm = jnp.where(pos < lens_b, seqs.astype(jnp.float32),
                   float(CHAR_VOCAB)).astype(jnp.bfloat16)
    col = lax.broadcasted_iota(jnp.int32, (L, L * CHAR_VOCAB), 1)
    row = lax.broadcasted_iota(jnp.int32, (L, L * CHAR_VOCAB), 0)
    p = (col // CHAR_VOCAB == row).astype(jnp.bfloat16)
    ch_all = jnp.dot(sm, p,
                     preferred_element_type=jnp.float32)
    cvec = lax.broadcasted_iota(
        jnp.int32, (1, CHAR_VOCAB), 1).astype(jnp.float32)
    counts_bf = jnp.zeros((V_BLOCK, CHAR_VOCAB), jnp.bfloat16)
    for l in range(L):
        sl = ch_all[:, l * CHAR_VOCAB:(l + 1) * CHAR_VOCAB]
        counts_bf = counts_bf + (sl == cvec).astype(jnp.bfloat16)
    counts = counts_bf.astype(jnp.float32)
    inv1 = 1.0 / lens_f
    inv_b = jnp.dot(inv1, jnp.ones((1, CHAR_DIM), jnp.float32),
                    preferred_element_type=jnp.float32)
    pooled = jnp.dot(counts, ce_ref[...],
                     preferred_element_type=jnp.float32) * inv_b
    out_ref[...] = jnp.dot(pooled, w_ref[...],
                           preferred_element_type=jnp.float32) + b_ref[...]


@functools.lru_cache(maxsize=None)
def _make_table_builder(vocab):
    grid = vocab // V_BLOCK
    return pl.pallas_call(
        _table_body,
        grid=(grid,),
        in_specs=[
            pl.BlockSpec((V_BLOCK, MAX_WORD_LEN), lambda i: (i, 0)),
            pl.BlockSpec((V_BLOCK, 1), lambda i: (i, 0)),
            pl.BlockSpec((CHAR_VOCAB, CHAR_DIM), lambda i: (0, 0)),
            pl.BlockSpec((CHAR_DIM, EMBED_SIZE), lambda i: (0, 0)),
            pl.BlockSpec((1, EMBED_SIZE), lambda i: (0, 0)),
        ],
        out_specs=pl.BlockSpec((V_BLOCK, EMBED_SIZE), lambda i: (i, 0)),
        out_shape=jax.ShapeDtypeStruct((vocab, EMBED_SIZE), jnp.float32),
    )


@functools.lru_cache(maxsize=None)
def _make_gather(n, vocab):
    info = plsc.get_sparse_core_info()
    nw = info.num_cores * info.num_subcores
    assert n % (nw * CHUNK) == 0
    n_chunks = n // (nw * CHUNK)
    mesh = plsc.VectorSubcoreMesh(core_axis_name="c", subcore_axis_name="s")

    @functools.partial(
        pl.kernel,
        mesh=mesh,
        out_type=jax.ShapeDtypeStruct((n, EMBED_SIZE), jnp.float32),
        scratch_types=[
            pltpu.VMEM((n_chunks, CHUNK), jnp.int32),
            pltpu.VMEM((CHUNK, EMBED_SIZE), jnp.float32),
            pltpu.SemaphoreType.DMA,
        ],
    )
    def gather_kernel(idx_hbm, table_hbm, out_hbm, idx_v, rows_v, sem):
        wid = lax.axis_index("s") * info.num_cores + lax.axis_index("c")
        row_base = wid * n_chunks
        pltpu.sync_copy(idx_hbm.at[wid], idx_v)

        def chunk_body(c, carry):
            pltpu.async_copy(table_hbm.at[idx_v.at[c]], rows_v, sem).wait()
            pltpu.sync_copy(
                rows_v, out_hbm.at[pl.ds((row_base + c) * CHUNK, CHUNK)])
            return carry

        lax.fori_loop(0, n_chunks, chunk_body, 0)

    def run(ids_flat, table):
        return gather_kernel(ids_flat.reshape(nw, n_chunks, CHUNK), table)

    return run


def kernel(word_ids, sequences, sequences_length, char_embed, W, b):
    shape = word_ids.shape
    n = word_ids.size
    vocab = sequences.shape[0]
    ids = jnp.asarray(word_ids, jnp.int32).reshape(-1)
    seqs = jnp.asarray(sequences, jnp.int32)
    lens = jnp.asarray(sequences_length, jnp.int32).reshape(vocab, 1)
    table = _make_table_builder(vocab)(
        seqs, lens, char_embed, W, b.reshape(1, EMBED_SIZE))
    out = _make_gather(n, vocab)(ids, table)
    return out.reshape(shape + (EMBED_SIZE,))


# speedup vs baseline: 40.7669x; 1.3922x over previous
import functools

import jax
import jax.numpy as jnp
from jax import lax
from jax.experimental import pallas as pl
from jax.experimental.pallas import tpu as pltpu
from jax.experimental.pallas import tpu_sc as plsc

MAX_WORD_LEN = 16
CHAR_VOCAB = 128
CHAR_DIM = 64
EMBED_SIZE = 128

V_BLOCK = 2000
CHUNK = 128


def _table_body(seqs_ref, lens_ref, ce_ref, w_ref, b_ref, out_ref):
    L = MAX_WORD_LEN
    seqs = seqs_ref[...]
    lens = lens_ref[...]
    lens_f = lens.astype(jnp.float32)
    ones_l = jnp.ones((1, L), jnp.float32)
    lens_b = jnp.dot(lens_f, ones_l,
                     preferred_element_type=jnp.float32)
    pos = lax.broadcasted_iota(jnp.int32, (1, L), 1).astype(jnp.float32)
    sm = jnp.where(pos < lens_b, seqs.astype(jnp.float32),
                   float(CHAR_VOCAB)).astype(jnp.bfloat16)
    col = lax.broadcasted_iota(jnp.int32, (L, L * CHAR_VOCAB), 1)
    row = lax.broadcasted_iota(jnp.int32, (L, L * CHAR_VOCAB), 0)
    p = (col // CHAR_VOCAB == row).astype(jnp.bfloat16)
    ch_all = jnp.dot(sm, p,
                     preferred_element_type=jnp.float32)
    cvec = lax.broadcasted_iota(
        jnp.int32, (1, CHAR_VOCAB), 1).astype(jnp.float32)
    counts_bf = jnp.zeros((V_BLOCK, CHAR_VOCAB), jnp.bfloat16)
    for l in range(L):
        sl = ch_all[:, l * CHAR_VOCAB:(l + 1) * CHAR_VOCAB]
        counts_bf = counts_bf + (sl == cvec).astype(jnp.bfloat16)
    counts = counts_bf.astype(jnp.float32)
    inv1 = 1.0 / lens_f
    inv_b = jnp.dot(inv1, jnp.ones((1, CHAR_DIM), jnp.float32),
                    preferred_element_type=jnp.float32)
    pooled = jnp.dot(counts, ce_ref[...],
                     preferred_element_type=jnp.float32) * inv_b
    out_ref[...] = jnp.dot(pooled, w_ref[...],
                           preferred_element_type=jnp.float32) + b_ref[...]


@functools.lru_cache(maxsize=None)
def _make_table_builder(vocab):
    grid = vocab // V_BLOCK
    return pl.pallas_call(
        _table_body,
        grid=(grid,),
        in_specs=[
            pl.BlockSpec((V_BLOCK, MAX_WORD_LEN), lambda i: (i, 0)),
            pl.BlockSpec((V_BLOCK, 1), lambda i: (i, 0)),
            pl.BlockSpec((CHAR_VOCAB, CHAR_DIM), lambda i: (0, 0)),
            pl.BlockSpec((CHAR_DIM, EMBED_SIZE), lambda i: (0, 0)),
            pl.BlockSpec((1, EMBED_SIZE), lambda i: (0, 0)),
        ],
        out_specs=pl.BlockSpec((V_BLOCK, EMBED_SIZE), lambda i: (i, 0)),
        out_shape=jax.ShapeDtypeStruct((vocab, EMBED_SIZE), jnp.float32),
    )


@functools.lru_cache(maxsize=None)
def _make_gather(batch, hist, vocab):
    info = plsc.get_sparse_core_info()
    nw = info.num_cores * info.num_subcores
    group = 8
    assert batch % (nw * group) == 0
    rows_per_w = batch // nw
    n_groups = rows_per_w // group
    mesh = plsc.VectorSubcoreMesh(core_axis_name="c", subcore_axis_name="s")

    @functools.partial(
        pl.kernel,
        mesh=mesh,
        out_type=jax.ShapeDtypeStruct((batch, hist, EMBED_SIZE), jnp.float32),
        scratch_types=[
            pltpu.VMEM((rows_per_w, hist), jnp.int32),
            pltpu.VMEM((group, hist, EMBED_SIZE), jnp.float32),
            pltpu.SemaphoreType.DMA,
        ],
    )
    def gather_kernel(idx_hbm, table_hbm, out_hbm, idx_v, rows_v, sem):
        wid = lax.axis_index("s") * info.num_cores + lax.axis_index("c")
        base = wid * rows_per_w
        pltpu.sync_copy(idx_hbm.at[wid], idx_v)

        def group_body(g, carry):
            copies = [
                pltpu.async_copy(
                    table_hbm.at[idx_v.at[g * group + j]], rows_v.at[j], sem)
                for j in range(group)
            ]
            for c in copies:
                c.wait()
            pltpu.sync_copy(rows_v, out_hbm.at[pl.ds(base + g * group, group)])
            return carry

        lax.fori_loop(0, n_groups, group_body, 0)

    def run(ids, table):
        return gather_kernel(ids.reshape(nw, rows_per_w, hist), table)

    return run


def kernel(word_ids, sequences, sequences_length, char_embed, W, b):
    batch, hist = word_ids.shape
    vocab = sequences.shape[0]
    ids = jnp.asarray(word_ids, jnp.int32)
    seqs = jnp.asarray(sequences, jnp.int32)
    lens = jnp.asarray(sequences_length, jnp.int32).reshape(vocab, 1)
    table = _make_table_builder(vocab)(
        seqs, lens, char_embed, W, b.reshape(1, EMBED_SIZE))
    return _make_gather(batch, hist, vocab)(ids, table)


# speedup vs baseline: 40.7703x; 1.0001x over previous
import functools

import jax
import jax.numpy as jnp
from jax import lax
from jax.experimental import pallas as pl
from jax.experimental.pallas import tpu as pltpu
from jax.experimental.pallas import tpu_sc as plsc

MAX_WORD_LEN = 16
CHAR_VOCAB = 128
CHAR_DIM = 64
EMBED_SIZE = 128

V_BLOCK = 2000
CHUNK = 128


def _table_body(seqs_ref, lens_ref, ce_ref, w_ref, b_ref, out_ref):
    L = MAX_WORD_LEN
    seqs = seqs_ref[...]
    lens = lens_ref[...]
    lens_f = lens.astype(jnp.float32)
    ones_l = jnp.ones((1, L), jnp.float32)
    lens_b = jnp.dot(lens_f, ones_l,
                     preferred_element_type=jnp.float32)
    pos = lax.broadcasted_iota(jnp.int32, (1, L), 1).astype(jnp.float32)
    sm = jnp.where(pos < lens_b, seqs.astype(jnp.float32),
                   float(CHAR_VOCAB)).astype(jnp.bfloat16)
    col = lax.broadcasted_iota(jnp.int32, (L, L * CHAR_VOCAB), 1)
    row = lax.broadcasted_iota(jnp.int32, (L, L * CHAR_VOCAB), 0)
    p = (col // CHAR_VOCAB == row).astype(jnp.bfloat16)
    ch_all = jnp.dot(sm, p,
                     preferred_element_type=jnp.float32)
    cvec = lax.broadcasted_iota(
        jnp.int32, (1, CHAR_VOCAB), 1).astype(jnp.float32)
    counts_bf = jnp.zeros((V_BLOCK, CHAR_VOCAB), jnp.bfloat16)
    for l in range(L):
        sl = ch_all[:, l * CHAR_VOCAB:(l + 1) * CHAR_VOCAB]
        counts_bf = counts_bf + (sl == cvec).astype(jnp.bfloat16)
    counts = counts_bf.astype(jnp.float32)
    inv1 = 1.0 / lens_f
    inv_b = jnp.dot(inv1, jnp.ones((1, CHAR_DIM), jnp.float32),
                    preferred_element_type=jnp.float32)
    pooled = jnp.dot(counts, ce_ref[...],
                     preferred_element_type=jnp.float32) * inv_b
    out_ref[...] = jnp.dot(pooled, w_ref[...],
                           preferred_element_type=jnp.float32) + b_ref[...]


@functools.lru_cache(maxsize=None)
def _make_table_builder(vocab):
    grid = vocab // V_BLOCK
    return pl.pallas_call(
        _table_body,
        grid=(grid,),
        in_specs=[
            pl.BlockSpec((V_BLOCK, MAX_WORD_LEN), lambda i: (i, 0)),
            pl.BlockSpec((V_BLOCK, 1), lambda i: (i, 0)),
            pl.BlockSpec((CHAR_VOCAB, CHAR_DIM), lambda i: (0, 0)),
            pl.BlockSpec((CHAR_DIM, EMBED_SIZE), lambda i: (0, 0)),
            pl.BlockSpec((1, EMBED_SIZE), lambda i: (0, 0)),
        ],
        out_specs=pl.BlockSpec((V_BLOCK, EMBED_SIZE), lambda i: (i, 0)),
        out_shape=jax.ShapeDtypeStruct((vocab, EMBED_SIZE), jnp.float32),
    )


@functools.lru_cache(maxsize=None)
def _make_gather(batch, hist, vocab):
    info = plsc.get_sparse_core_info()
    nw = info.num_cores * info.num_subcores
    group = 8
    assert batch % (nw * group) == 0
    rows_per_w = batch // nw
    n_groups = rows_per_w // group
    mesh = plsc.VectorSubcoreMesh(core_axis_name="c", subcore_axis_name="s")

    @functools.partial(
        pl.kernel,
        mesh=mesh,
        out_type=jax.ShapeDtypeStruct((batch, hist, EMBED_SIZE), jnp.float32),
        scratch_types=[
            pltpu.VMEM((rows_per_w, hist), jnp.int32),
            pltpu.VMEM((group, hist, EMBED_SIZE), jnp.float32),
            pltpu.SemaphoreType.DMA,
        ],
    )
    def gather_kernel(idx_hbm, table_hbm, out_hbm, idx_v, rows_v, sem):
        wid = lax.axis_index("s") * info.num_cores + lax.axis_index("c")
        base = wid * rows_per_w
        pltpu.sync_copy(idx_hbm.at[pl.ds(base, rows_per_w)], idx_v)

        def group_body(g, carry):
            copies = [
                pltpu.async_copy(
                    table_hbm.at[idx_v.at[g * group + j]], rows_v.at[j], sem)
                for j in range(group)
            ]
            for c in copies:
                c.wait()
            pltpu.sync_copy(rows_v, out_hbm.at[pl.ds(base + g * group, group)])
            return carry

        lax.fori_loop(0, n_groups, group_body, 0)

    return gather_kernel


def kernel(word_ids, sequences, sequences_length, char_embed, W, b):
    batch, hist = word_ids.shape
    vocab = sequences.shape[0]
    ids = jnp.asarray(word_ids, jnp.int32)
    seqs = jnp.asarray(sequences, jnp.int32)
    lens = jnp.asarray(sequences_length, jnp.int32).reshape(vocab, 1)
    table = _make_table_builder(vocab)(
        seqs, lens, char_embed, W, b.reshape(1, EMBED_SIZE))
    return _make_gather(batch, hist, vocab)(ids, table)


# speedup vs baseline: 40.7738x; 1.0001x over previous
import functools

import jax
import jax.numpy as jnp
from jax import lax
from jax.experimental import pallas as pl
from jax.experimental.pallas import tpu as pltpu
from jax.experimental.pallas import tpu_sc as plsc

MAX_WORD_LEN = 16
CHAR_VOCAB = 128
CHAR_DIM = 64
EMBED_SIZE = 128

V_BLOCK = 2000
CHUNK = 128


def _table_body(seqs_ref, lens_ref, ce_ref, w_ref, b_ref, out_ref):
    L = MAX_WORD_LEN
    seqs = seqs_ref[...]
    lens = lens_ref[...]
    lens_f = lens.astype(jnp.float32)
    ones_l = jnp.ones((1, L), jnp.float32)
    lens_b = jnp.dot(lens_f, ones_l,
                     preferred_element_type=jnp.float32)
    pos = lax.broadcasted_iota(jnp.int32, (1, L), 1).astype(jnp.float32)
    sm = jnp.where(pos < lens_b, seqs.astype(jnp.float32),
                   float(CHAR_VOCAB)).astype(jnp.bfloat16)
    col = lax.broadcasted_iota(jnp.int32, (L, L * CHAR_VOCAB), 1)
    row = lax.broadcasted_iota(jnp.int32, (L, L * CHAR_VOCAB), 0)
    p = (col // CHAR_VOCAB == row).astype(jnp.bfloat16)
    ch_all = jnp.dot(sm, p,
                     preferred_element_type=jnp.float32)
    cvec = lax.broadcasted_iota(
        jnp.int32, (1, CHAR_VOCAB), 1).astype(jnp.float32)
    counts_bf = jnp.zeros((V_BLOCK, CHAR_VOCAB), jnp.bfloat16)
    for l in range(L):
        sl = ch_all[:, l * CHAR_VOCAB:(l + 1) * CHAR_VOCAB]
        counts_bf = counts_bf + (sl == cvec).astype(jnp.bfloat16)
    counts = counts_bf.astype(jnp.float32)
    inv1 = 1.0 / lens_f
    inv_b = jnp.dot(inv1, jnp.ones((1, CHAR_DIM), jnp.float32),
                    preferred_element_type=jnp.float32)
    pooled = jnp.dot(counts, ce_ref[...],
                     preferred_element_type=jnp.float32) * inv_b
    out_ref[...] = jnp.dot(pooled, w_ref[...],
                           preferred_element_type=jnp.float32) + b_ref[...]


@functools.lru_cache(maxsize=None)
def _make_table_builder(vocab):
    grid = vocab // V_BLOCK
    return pl.pallas_call(
        _table_body,
        grid=(grid,),
        in_specs=[
            pl.BlockSpec((V_BLOCK, MAX_WORD_LEN), lambda i: (i, 0)),
            pl.BlockSpec((V_BLOCK, 1), lambda i: (i, 0)),
            pl.BlockSpec((CHAR_VOCAB, CHAR_DIM), lambda i: (0, 0)),
            pl.BlockSpec((CHAR_DIM, EMBED_SIZE), lambda i: (0, 0)),
            pl.BlockSpec((1, EMBED_SIZE), lambda i: (0, 0)),
        ],
        out_specs=pl.BlockSpec((V_BLOCK, EMBED_SIZE), lambda i: (i, 0)),
        out_shape=jax.ShapeDtypeStruct((vocab, EMBED_SIZE), jnp.float32),
    )


@functools.lru_cache(maxsize=None)
def _make_gather(batch, hist, vocab):
    info = plsc.get_sparse_core_info()
    nw = info.num_cores * info.num_subcores
    group = 8
    assert batch % (nw * group) == 0
    rows_per_w = batch // nw
    n_groups = rows_per_w // group
    mesh = plsc.VectorSubcoreMesh(core_axis_name="c", subcore_axis_name="s")

    @functools.partial(
        pl.kernel,
        mesh=mesh,
        compiler_params=pltpu.CompilerParams(use_tc_tiling_on_sc=True),
        out_type=jax.ShapeDtypeStruct((batch, hist, EMBED_SIZE), jnp.float32),
        scratch_types=[
            pltpu.VMEM((rows_per_w, hist), jnp.int32),
            pltpu.VMEM((group, hist, EMBED_SIZE), jnp.float32),
            pltpu.SemaphoreType.DMA,
        ],
    )
    def gather_kernel(idx_hbm, table_hbm, out_hbm, idx_v, rows_v, sem):
        wid = lax.axis_index("s") * info.num_cores + lax.axis_index("c")
        base = wid * rows_per_w
        pltpu.sync_copy(idx_hbm.at[pl.ds(base, rows_per_w)], idx_v)

        def group_body(g, carry):
            copies = [
                pltpu.async_copy(
                    table_hbm.at[idx_v.at[g * group + j]], rows_v.at[j], sem)
                for j in range(group)
            ]
            for c in copies:
                c.wait()
            pltpu.sync_copy(rows_v, out_hbm.at[pl.ds(base + g * group, group)])
            return carry

        lax.fori_loop(0, n_groups, group_body, 0)

    return gather_kernel


def kernel(word_ids, sequences, sequences_length, char_embed, W, b):
    batch, hist = word_ids.shape
    vocab = sequences.shape[0]
    ids = jnp.asarray(word_ids, jnp.int32)
    seqs = jnp.asarray(sequences, jnp.int32)
    lens = jnp.asarray(sequences_length, jnp.int32).reshape(vocab, 1)
    table = _make_table_builder(vocab)(
        seqs, lens, char_embed, W, b.reshape(1, EMBED_SIZE))
    return _make_gather(batch, hist, vocab)(ids, table)


# speedup vs baseline: 50.9161x; 1.2487x over previous
import functools

import jax
import jax.numpy as jnp
from jax import lax
from jax.experimental import pallas as pl
from jax.experimental.pallas import tpu as pltpu
from jax.experimental.pallas import tpu_sc as plsc

MAX_WORD_LEN = 16
CHAR_VOCAB = 128
CHAR_DIM = 64
EMBED_SIZE = 128

V_BLOCK = 2000
CHUNK = 128


def _table_body(seqs_ref, lens_ref, ce_ref, w_ref, b_ref, out_ref):
    L = MAX_WORD_LEN
    seqs = seqs_ref[...]
    lens = lens_ref[...]
    lens_f = lens.astype(jnp.float32)
    ones_l = jnp.ones((1, L), jnp.float32)
    lens_b = jnp.dot(lens_f, ones_l,
                     preferred_element_type=jnp.float32)
    pos = lax.broadcasted_iota(jnp.int32, (1, L), 1).astype(jnp.float32)
    sm = jnp.where(pos < lens_b, seqs.astype(jnp.float32),
                   float(CHAR_VOCAB)).astype(jnp.bfloat16)
    col = lax.broadcasted_iota(jnp.int32, (L, L * CHAR_VOCAB), 1)
    row = lax.broadcasted_iota(jnp.int32, (L, L * CHAR_VOCAB), 0)
    p = (col // CHAR_VOCAB == row).astype(jnp.bfloat16)
    ch_all = jnp.dot(sm, p,
                     preferred_element_type=jnp.float32)
    cvec = lax.broadcasted_iota(
        jnp.int32, (1, CHAR_VOCAB), 1).astype(jnp.float32)
    counts_bf = jnp.zeros((V_BLOCK, CHAR_VOCAB), jnp.bfloat16)
    for l in range(L):
        sl = ch_all[:, l * CHAR_VOCAB:(l + 1) * CHAR_VOCAB]
        counts_bf = counts_bf + (sl == cvec).astype(jnp.bfloat16)
    counts = counts_bf.astype(jnp.float32)
    inv1 = 1.0 / lens_f
    inv_b = jnp.dot(inv1, jnp.ones((1, CHAR_DIM), jnp.float32),
                    preferred_element_type=jnp.float32)
    pooled = jnp.dot(counts, ce_ref[...],
                     preferred_element_type=jnp.float32) * inv_b
    out_ref[...] = jnp.dot(pooled, w_ref[...],
                           preferred_element_type=jnp.float32) + b_ref[...]


@functools.lru_cache(maxsize=None)
def _make_table_builder(vocab):
    grid = vocab // V_BLOCK
    return pl.pallas_call(
        _table_body,
        grid=(grid,),
        in_specs=[
            pl.BlockSpec((V_BLOCK, MAX_WORD_LEN), lambda i: (i, 0)),
            pl.BlockSpec((V_BLOCK, 1), lambda i: (i, 0)),
            pl.BlockSpec((CHAR_VOCAB, CHAR_DIM), lambda i: (0, 0)),
            pl.BlockSpec((CHAR_DIM, EMBED_SIZE), lambda i: (0, 0)),
            pl.BlockSpec((1, EMBED_SIZE), lambda i: (0, 0)),
        ],
        out_specs=pl.BlockSpec((V_BLOCK, EMBED_SIZE), lambda i: (i, 0)),
        out_shape=jax.ShapeDtypeStruct((vocab, EMBED_SIZE), jnp.float32),
    )


@functools.lru_cache(maxsize=None)
def _make_gather(batch, hist, vocab):
    info = plsc.get_sparse_core_info()
    nw = info.num_cores * info.num_subcores
    group = 5 if hist % 5 == 0 else 1
    n_groups = hist // group
    rows_per_w = batch // nw
    assert batch % nw == 0 and rows_per_w % 8 == 0
    mesh = plsc.VectorSubcoreMesh(core_axis_name="c", subcore_axis_name="s")

    @functools.partial(
        pl.kernel,
        mesh=mesh,
        out_type=jax.ShapeDtypeStruct((hist, batch, EMBED_SIZE), jnp.float32),
        scratch_types=[
            pltpu.VMEM((hist, rows_per_w), jnp.int32),
            pltpu.VMEM((group, rows_per_w, EMBED_SIZE), jnp.float32),
            pltpu.SemaphoreType.DMA,
        ],
    )
    def gather_kernel(idx_hbm, table_hbm, out_hbm, idx_v, rows_v, sem):
        wid = lax.axis_index("s") * info.num_cores + lax.axis_index("c")
        base = wid * rows_per_w
        pltpu.sync_copy(idx_hbm.at[:, pl.ds(base, rows_per_w)], idx_v)

        def group_body(g, carry):
            copies = [
                pltpu.async_copy(
                    table_hbm.at[idx_v.at[g * group + j]], rows_v.at[j], sem)
                for j in range(group)
            ]
            for c in copies:
                c.wait()
            pltpu.sync_copy(
                rows_v,
                out_hbm.at[pl.ds(g * group, group), pl.ds(base, rows_per_w)])
            return carry

        lax.fori_loop(0, n_groups, group_body, 0)

    return gather_kernel


def kernel(word_ids, sequences, sequences_length, char_embed, W, b):
    batch, hist = word_ids.shape
    vocab = sequences.shape[0]
    ids = jnp.asarray(word_ids, jnp.int32)
    seqs = jnp.asarray(sequences, jnp.int32)
    lens = jnp.asarray(sequences_length, jnp.int32).reshape(vocab, 1)
    table = _make_table_builder(vocab)(
        seqs, lens, char_embed, W, b.reshape(1, EMBED_SIZE))
    out_t = _make_gather(batch, hist, vocab)(ids.T, table)
    return jnp.transpose(out_t, (1, 0, 2))


# speedup vs baseline: 57.8867x; 1.1369x over previous
import functools

import jax
import jax.numpy as jnp
from jax import lax
from jax.experimental import pallas as pl
from jax.experimental.pallas import tpu as pltpu
from jax.experimental.pallas import tpu_sc as plsc

MAX_WORD_LEN = 16
CHAR_VOCAB = 128
CHAR_DIM = 64
EMBED_SIZE = 128

V_BLOCK = 2000
CHUNK = 128


def _table_body(seqs_ref, lens_ref, ce_ref, w_ref, b_ref, out_ref):
    L = MAX_WORD_LEN
    seqs = seqs_ref[...]
    lens = lens_ref[...].reshape(V_BLOCK, 1)
    lens_f = lens.astype(jnp.float32)
    ones_l = jnp.ones((1, L), jnp.float32)
    lens_b = jnp.dot(lens_f, ones_l,
                     preferred_element_type=jnp.float32)
    pos = lax.broadcasted_iota(jnp.int32, (1, L), 1).astype(jnp.float32)
    sm = jnp.where(pos < lens_b, seqs.astype(jnp.float32),
                   float(CHAR_VOCAB)).astype(jnp.bfloat16)
    col = lax.broadcasted_iota(jnp.int32, (L, L * CHAR_VOCAB), 1)
    row = lax.broadcasted_iota(jnp.int32, (L, L * CHAR_VOCAB), 0)
    p = (col // CHAR_VOCAB == row).astype(jnp.bfloat16)
    ch_all = jnp.dot(sm, p,
                     preferred_element_type=jnp.float32)
    cvec = lax.broadcasted_iota(
        jnp.int32, (1, CHAR_VOCAB), 1).astype(jnp.float32)
    counts_bf = jnp.zeros((V_BLOCK, CHAR_VOCAB), jnp.bfloat16)
    for l in range(L):
        sl = ch_all[:, l * CHAR_VOCAB:(l + 1) * CHAR_VOCAB]
        counts_bf = counts_bf + (sl == cvec).astype(jnp.bfloat16)
    counts = counts_bf.astype(jnp.float32)
    inv1 = 1.0 / lens_f
    inv_b = jnp.dot(inv1, jnp.ones((1, CHAR_DIM), jnp.float32),
                    preferred_element_type=jnp.float32)
    pooled = jnp.dot(counts, ce_ref[...],
                     preferred_element_type=jnp.float32) * inv_b
    out_ref[...] = jnp.dot(pooled, w_ref[...],
                           preferred_element_type=jnp.float32) + b_ref[...]


@functools.lru_cache(maxsize=None)
def _make_table_builder(vocab):
    grid = vocab // V_BLOCK
    return pl.pallas_call(
        _table_body,
        grid=(grid,),
        in_specs=[
            pl.BlockSpec((V_BLOCK, MAX_WORD_LEN), lambda i: (i, 0)),
            pl.BlockSpec((1, 1, V_BLOCK), lambda i: (i, 0, 0)),
            pl.BlockSpec((CHAR_VOCAB, CHAR_DIM), lambda i: (0, 0)),
            pl.BlockSpec((CHAR_DIM, EMBED_SIZE), lambda i: (0, 0)),
            pl.BlockSpec((1, EMBED_SIZE), lambda i: (0, 0)),
        ],
        out_specs=pl.BlockSpec((V_BLOCK, EMBED_SIZE), lambda i: (i, 0)),
        out_shape=jax.ShapeDtypeStruct((vocab, EMBED_SIZE), jnp.float32),
    )


@functools.lru_cache(maxsize=None)
def _make_gather(batch, hist, vocab):
    info = plsc.get_sparse_core_info()
    nw = info.num_cores * info.num_subcores
    group = 5 if hist % 5 == 0 else 1
    n_groups = hist // group
    rows_per_w = batch // nw
    assert batch % nw == 0 and rows_per_w % 8 == 0
    mesh = plsc.VectorSubcoreMesh(core_axis_name="c", subcore_axis_name="s")

    @functools.partial(
        pl.kernel,
        mesh=mesh,
        out_type=jax.ShapeDtypeStruct((hist, batch, EMBED_SIZE), jnp.float32),
        scratch_types=[
            pltpu.VMEM((hist, rows_per_w), jnp.int32),
            pltpu.VMEM((group, rows_per_w, EMBED_SIZE), jnp.float32),
            pltpu.SemaphoreType.DMA,
        ],
    )
    def gather_kernel(idx_hbm, table_hbm, out_hbm, idx_v, rows_v, sem):
        wid = lax.axis_index("s") * info.num_cores + lax.axis_index("c")
        base = wid * rows_per_w
        pltpu.sync_copy(idx_hbm.at[:, pl.ds(base, rows_per_w)], idx_v)

        def group_body(g, carry):
            copies = [
                pltpu.async_copy(
                    table_hbm.at[idx_v.at[g * group + j]], rows_v.at[j], sem)
                for j in range(group)
            ]
            for c in copies:
                c.wait()
            pltpu.sync_copy(
                rows_v,
                out_hbm.at[pl.ds(g * group, group), pl.ds(base, rows_per_w)])
            return carry

        lax.fori_loop(0, n_groups, group_body, 0)

    return gather_kernel


def kernel(word_ids, sequences, sequences_length, char_embed, W, b):
    batch, hist = word_ids.shape
    vocab = sequences.shape[0]
    ids = jnp.asarray(word_ids, jnp.int32)
    seqs = jnp.asarray(sequences, jnp.int32)
    lens = jnp.asarray(sequences_length, jnp.int32).reshape(
        vocab // V_BLOCK, 1, V_BLOCK)
    table = _make_table_builder(vocab)(
        seqs, lens, char_embed, W, b.reshape(1, EMBED_SIZE))
    out_t = _make_gather(batch, hist, vocab)(ids.T, table)
    return jnp.transpose(out_t, (1, 0, 2))


# speedup vs baseline: 59.5563x; 1.0288x over previous
import functools

import jax
import jax.numpy as jnp
from jax import lax
from jax.experimental import pallas as pl
from jax.experimental.pallas import tpu as pltpu
from jax.experimental.pallas import tpu_sc as plsc

MAX_WORD_LEN = 16
CHAR_VOCAB = 128
CHAR_DIM = 64
EMBED_SIZE = 128

V_BLOCK = 4000
CHUNK = 128


def _table_body(seqs_ref, lens_ref, ce_ref, w_ref, b_ref, out_ref):
    L = MAX_WORD_LEN
    seqs = seqs_ref[...]
    lens = lens_ref[...].reshape(V_BLOCK, 1)
    lens_f = lens.astype(jnp.float32)
    ones_l = jnp.ones((1, L), jnp.float32)
    lens_b = jnp.dot(lens_f, ones_l,
                     preferred_element_type=jnp.float32)
    pos = lax.broadcasted_iota(jnp.int32, (1, L), 1).astype(jnp.float32)
    sm = jnp.where(pos < lens_b, seqs.astype(jnp.float32),
                   float(CHAR_VOCAB)).astype(jnp.bfloat16)
    col = lax.broadcasted_iota(jnp.int32, (L, L * CHAR_VOCAB), 1)
    row = lax.broadcasted_iota(jnp.int32, (L, L * CHAR_VOCAB), 0)
    p = (col // CHAR_VOCAB == row).astype(jnp.bfloat16)
    ch_all = jnp.dot(sm, p,
                     preferred_element_type=jnp.float32)
    cvec = lax.broadcasted_iota(
        jnp.int32, (1, CHAR_VOCAB), 1).astype(jnp.float32)
    counts_bf = jnp.zeros((V_BLOCK, CHAR_VOCAB), jnp.bfloat16)
    for l in range(L):
        sl = ch_all[:, l * CHAR_VOCAB:(l + 1) * CHAR_VOCAB]
        counts_bf = counts_bf + (sl == cvec).astype(jnp.bfloat16)
    counts = counts_bf.astype(jnp.float32)
    inv1 = 1.0 / lens_f
    inv_b = jnp.dot(inv1, jnp.ones((1, CHAR_DIM), jnp.float32),
                    preferred_element_type=jnp.float32)
    pooled = jnp.dot(counts, ce_ref[...],
                     preferred_element_type=jnp.float32) * inv_b
    out_ref[...] = jnp.dot(pooled, w_ref[...],
                           preferred_element_type=jnp.float32) + b_ref[...]


@functools.lru_cache(maxsize=None)
def _make_table_builder(vocab):
    grid = vocab // V_BLOCK
    return pl.pallas_call(
        _table_body,
        grid=(grid,),
        in_specs=[
            pl.BlockSpec((V_BLOCK, MAX_WORD_LEN), lambda i: (i, 0)),
            pl.BlockSpec((1, 1, V_BLOCK), lambda i: (i, 0, 0)),
            pl.BlockSpec((CHAR_VOCAB, CHAR_DIM), lambda i: (0, 0)),
            pl.BlockSpec((CHAR_DIM, EMBED_SIZE), lambda i: (0, 0)),
            pl.BlockSpec((1, EMBED_SIZE), lambda i: (0, 0)),
        ],
        out_specs=pl.BlockSpec((V_BLOCK, EMBED_SIZE), lambda i: (i, 0)),
        out_shape=jax.ShapeDtypeStruct((vocab, EMBED_SIZE), jnp.float32),
    )


@functools.lru_cache(maxsize=None)
def _make_gather(batch, hist, vocab):
    info = plsc.get_sparse_core_info()
    nw = info.num_cores * info.num_subcores
    group = 5 if hist % 5 == 0 else 1
    n_groups = hist // group
    rows_per_w = batch // nw
    assert batch % nw == 0 and rows_per_w % 8 == 0
    mesh = plsc.VectorSubcoreMesh(core_axis_name="c", subcore_axis_name="s")

    @functools.partial(
        pl.kernel,
        mesh=mesh,
        out_type=jax.ShapeDtypeStruct((hist, batch, EMBED_SIZE), jnp.float32),
        scratch_types=[
            pltpu.VMEM((hist, rows_per_w), jnp.int32),
            pltpu.VMEM((group, rows_per_w, EMBED_SIZE), jnp.float32),
            pltpu.SemaphoreType.DMA,
        ],
    )
    def gather_kernel(idx_hbm, table_hbm, out_hbm, idx_v, rows_v, sem):
        wid = lax.axis_index("s") * info.num_cores + lax.axis_index("c")
        base = wid * rows_per_w
        pltpu.sync_copy(idx_hbm.at[:, pl.ds(base, rows_per_w)], idx_v)

        def group_body(g, carry):
            copies = [
                pltpu.async_copy(
                    table_hbm.at[idx_v.at[g * group + j]], rows_v.at[j], sem)
                for j in range(group)
            ]
            for c in copies:
                c.wait()
            pltpu.sync_copy(
                rows_v,
                out_hbm.at[pl.ds(g * group, group), pl.ds(base, rows_per_w)])
            return carry

        lax.fori_loop(0, n_groups, group_body, 0)

    return gather_kernel


def kernel(word_ids, sequences, sequences_length, char_embed, W, b):
    batch, hist = word_ids.shape
    vocab = sequences.shape[0]
    ids = jnp.asarray(word_ids, jnp.int32)
    seqs = jnp.asarray(sequences, jnp.int32)
    lens = jnp.asarray(sequences_length, jnp.int32).reshape(
        vocab // V_BLOCK, 1, V_BLOCK)
    table = _make_table_builder(vocab)(
        seqs, lens, char_embed, W, b.reshape(1, EMBED_SIZE))
    out_t = _make_gather(batch, hist, vocab)(ids.T, table)
    return jnp.transpose(out_t, (1, 0, 2))
